# Initial kernel scaffold; baseline (speedup 1.0000x reference)
import jax
import jax.numpy as jnp
from jax.experimental import pallas as pl


def kernel(features, edge_index, W, b, a1_w, a1_b, a2_w, a2_b):
    raise NotImplementedError("write your pallas kernel here")


# speedup vs baseline: 28.4348x; 28.4348x over previous
import functools

import jax
import jax.numpy as jnp
from jax import lax
from jax.experimental import pallas as pl
from jax.experimental.pallas import tpu as pltpu
from jax.experimental.pallas import tpu_sc as plsc

N = 10000
NP = 10240
D = 128
TW = 144
E = 320000
NC = 2
NS = 16
EW = E // (NC * NS)
B = 80
NB = EW // B
RPT = NP // NS


def _prep_body(f_ref, wt_ref, b_ref, a2w_ref, a2b_ref, t_ref):
    h = jnp.dot(f_ref[...], wt_ref[...], preferred_element_type=jnp.float32)
    h = h + b_ref[...]
    a2 = jnp.sum(h * a2w_ref[...], axis=1, keepdims=True) + a2b_ref[0, 0]
    g = jnp.exp(a2)
    pad = jnp.zeros((h.shape[0], TW - D - 1), jnp.float32)
    t_ref[...] = jnp.concatenate([h * g, g, pad], axis=1)


def _make_table(features, wt, bvec, a2w, a2b):
    bm = 2000
    grid = (N // bm,)
    return pl.pallas_call(
        _prep_body,
        grid=grid,
        in_specs=[
            pl.BlockSpec((bm, D), lambda i: (i, 0)),
            pl.BlockSpec((D, D), lambda i: (0, 0)),
            pl.BlockSpec((1, D), lambda i: (0, 0)),
            pl.BlockSpec((1, D), lambda i: (0, 0)),
            pl.BlockSpec((1, 1), lambda i: (0, 0)),
        ],
        out_specs=pl.BlockSpec((bm, TW), lambda i: (i, 0)),
        out_shape=jax.ShapeDtypeStruct((NP, TW), jnp.float32),
    )(features, wt, bvec, a2w, a2b)


def _edge_body(t_hbm, src_hbm, dst_hbm, out_hbm, srcb, dstb, rows, acc, sem):
    cid = lax.axis_index("c")
    sid = lax.axis_index("s")

    def zb(i, _):
        for k in range(TW // 16):
            rows[i, pl.ds(k * 16, 16)] = jnp.zeros((16,), jnp.float32)
        return 0

    lax.fori_loop(0, B, zb, 0)
    for j in range(RPT // B):
        pltpu.sync_copy(rows, acc.at[pl.ds(sid * RPT + j * B, B)])
    plsc.subcore_barrier()

    base0 = (cid * NS + sid) * EW

    def bb(bi, _):
        base = base0 + bi * B
        pltpu.sync_copy(src_hbm.at[pl.ds(base, B)], srcb)
        pltpu.sync_copy(dst_hbm.at[pl.ds(base, B)], dstb)
        pltpu.async_copy(t_hbm.at[dstb], rows, sem).wait()
        pltpu.sync_copy(rows, acc.at[srcb], add=True)
        return 0

    lax.fori_loop(0, NB, bb, 0)
    plsc.subcore_barrier()
    pltpu.sync_copy(
        acc.at[pl.ds(sid * RPT, RPT)],
        out_hbm.at[cid, pl.ds(sid * RPT, RPT)],
    )


def _edge_accumulate(table, src, dst):
    mesh = plsc.VectorSubcoreMesh(core_axis_name="c", subcore_axis_name="s")
    f = pl.kernel(
        _edge_body,
        out_type=jax.ShapeDtypeStruct((NC, NP, TW), jnp.float32),
        mesh=mesh,
        scratch_types=[
            pltpu.VMEM((B,), jnp.int32),
            pltpu.VMEM((B,), jnp.int32),
            pltpu.VMEM((B, TW), jnp.float32),
            pltpu.VMEM_SHARED((NP, TW), jnp.float32),
            pltpu.SemaphoreType.DMA,
        ],
        compiler_params=pltpu.CompilerParams(use_tc_tiling_on_sc=False),
    )
    return f(table, src, dst)


def _comb_body(acc_ref, o_ref):
    a = acc_ref[0] + acc_ref[1]
    den = a[:, D:D + 1]
    den = jnp.where(den > 0.0, den, 1.0)
    o_ref[...] = a[:, :D] / den


def _combine(acc):
    bm = 2000
    grid = (N // bm,)
    return pl.pallas_call(
        _comb_body,
        grid=grid,
        in_specs=[pl.BlockSpec((NC, bm, TW), lambda i: (0, i, 0))],
        out_specs=pl.BlockSpec((bm, D), lambda i: (i, 0)),
        out_shape=jax.ShapeDtypeStruct((N, D), jnp.float32),
    )(acc)


@jax.jit
def kernel(features, edge_index, W, b, a1_w, a1_b, a2_w, a2_b):
    del a1_w, a1_b
    table = _make_table(
        features,
        W.T,
        b.reshape(1, D),
        a2_w.reshape(1, D),
        a2_b.reshape(1, 1),
    )
    acc = _edge_accumulate(table, edge_index[0], edge_index[1])
    return _combine(acc)


# speedup vs baseline: 42.1158x; 1.4811x over previous
import functools

import jax
import jax.numpy as jnp
from jax import lax
from jax.experimental import pallas as pl
from jax.experimental.pallas import tpu as pltpu
from jax.experimental.pallas import tpu_sc as plsc

N = 10000
NP = 10240
D = 128
TW = 144
E = 320000
NC = 2
NS = 16
EW = E // (NC * NS)
B = 80
NB = EW // B
RPT = NP // NS


def _prep_body(f_ref, wt_ref, b_ref, a2w_ref, a2b_ref, t_ref):
    h = jnp.dot(f_ref[...], wt_ref[...], preferred_element_type=jnp.float32)
    h = h + b_ref[...]
    a2 = jnp.sum(h * a2w_ref[...], axis=1, keepdims=True) + a2b_ref[0, 0]
    g = jnp.exp(a2)
    pad = jnp.zeros((h.shape[0], TW - D - 1), jnp.float32)
    t_ref[...] = jnp.concatenate([h * g, g, pad], axis=1)


def _make_table(features, wt, bvec, a2w, a2b):
    bm = 2000
    grid = (N // bm,)
    return pl.pallas_call(
        _prep_body,
        grid=grid,
        in_specs=[
            pl.BlockSpec((bm, D), lambda i: (i, 0)),
            pl.BlockSpec((D, D), lambda i: (0, 0)),
            pl.BlockSpec((1, D), lambda i: (0, 0)),
            pl.BlockSpec((1, D), lambda i: (0, 0)),
            pl.BlockSpec((1, 1), lambda i: (0, 0)),
        ],
        out_specs=pl.BlockSpec((bm, TW), lambda i: (i, 0)),
        out_shape=jax.ShapeDtypeStruct((NP, TW), jnp.float32),
    )(features, wt, bvec, a2w, a2b)


NPAIR = NB // 2


def _edge_body(t_hbm, src_hbm, dst_hbm, out_hbm,
               s0, d0, r0, s1, d1, r1, acc, g0, g1, t0, t1):
    cid = lax.axis_index("c")
    sid = lax.axis_index("s")
    srcb = (s0, s1)
    dstb = (d0, d1)
    rows = (r0, r1)
    gsem = (g0, g1)
    ssem = (t0, t1)

    def zb(i, _):
        for k in range(TW // 16):
            r0[i, pl.ds(k * 16, 16)] = jnp.zeros((16,), jnp.float32)
        return 0

    lax.fori_loop(0, B, zb, 0)
    for j in range(RPT // B):
        pltpu.sync_copy(r0, acc.at[pl.ds(sid * RPT + j * B, B)])
    plsc.subcore_barrier()

    base0 = (cid * NS + sid) * EW

    def idx_and_gather(bi, b):
        base = base0 + bi * B
        pltpu.sync_copy(src_hbm.at[pl.ds(base, B)], srcb[b])
        pltpu.sync_copy(dst_hbm.at[pl.ds(base, B)], dstb[b])
        pltpu.async_copy(t_hbm.at[dstb[b]], rows[b], gsem[b])

    def wait_gather(b):
        pltpu.make_async_copy(t_hbm.at[dstb[b]], rows[b], gsem[b]).wait()

    def start_scatter(b):
        pltpu.async_copy(rows[b], acc.at[srcb[b]], ssem[b], add=True)

    def wait_scatter(b):
        pltpu.make_async_copy(rows[b], acc.at[srcb[b]], ssem[b]).wait()

    idx_and_gather(0, 0)

    def pair(pi, _):
        i0 = pi * 2
        @pl.when(pi > 0)
        def _():
            wait_scatter(1)
        idx_and_gather(i0 + 1, 1)
        wait_gather(0)
        start_scatter(0)
        wait_scatter(0)
        @pl.when(pi < NPAIR - 1)
        def _():
            idx_and_gather(i0 + 2, 0)
        wait_gather(1)
        start_scatter(1)
        return 0

    lax.fori_loop(0, NPAIR, pair, 0)
    wait_scatter(1)
    idx_and_gather(NB - 1, 0)
    wait_gather(0)
    pltpu.sync_copy(r0, acc.at[s0], add=True)

    plsc.subcore_barrier()
    pltpu.sync_copy(
        acc.at[pl.ds(sid * RPT, RPT)],
        out_hbm.at[cid, pl.ds(sid * RPT, RPT)],
    )


def _edge_accumulate(table, src, dst):
    mesh = plsc.VectorSubcoreMesh(core_axis_name="c", subcore_axis_name="s")
    f = pl.kernel(
        _edge_body,
        out_type=jax.ShapeDtypeStruct((NC, NP, TW), jnp.float32),
        mesh=mesh,
        scratch_types=[
            pltpu.VMEM((B,), jnp.int32),
            pltpu.VMEM((B,), jnp.int32),
            pltpu.VMEM((B, TW), jnp.float32),
            pltpu.VMEM((B,), jnp.int32),
            pltpu.VMEM((B,), jnp.int32),
            pltpu.VMEM((B, TW), jnp.float32),
            pltpu.VMEM_SHARED((NP, TW), jnp.float32),
            pltpu.SemaphoreType.DMA,
            pltpu.SemaphoreType.DMA,
            pltpu.SemaphoreType.DMA,
            pltpu.SemaphoreType.DMA,
        ],
        compiler_params=pltpu.CompilerParams(use_tc_tiling_on_sc=False),
    )
    return f(table, src, dst)


def _comb_body(acc_ref, o_ref):
    a = acc_ref[0] + acc_ref[1]
    den = a[:, D:D + 1]
    den = jnp.where(den > 0.0, den, 1.0)
    o_ref[...] = a[:, :D] / den


def _combine(acc):
    bm = 2000
    grid = (N // bm,)
    return pl.pallas_call(
        _comb_body,
        grid=grid,
        in_specs=[pl.BlockSpec((NC, bm, TW), lambda i: (0, i, 0))],
        out_specs=pl.BlockSpec((bm, D), lambda i: (i, 0)),
        out_shape=jax.ShapeDtypeStruct((N, D), jnp.float32),
    )(acc)


@jax.jit
def kernel(features, edge_index, W, b, a1_w, a1_b, a2_w, a2_b):
    del a1_w, a1_b
    table = _make_table(
        features,
        W.T,
        b.reshape(1, D),
        a2_w.reshape(1, D),
        a2_b.reshape(1, 1),
    )
    acc = _edge_accumulate(table, edge_index[0], edge_index[1])
    return _combine(acc)


# speedup vs baseline: 52.6688x; 1.2506x over previous
import functools

import jax
import jax.numpy as jnp
from jax import lax
from jax.experimental import pallas as pl
from jax.experimental.pallas import tpu as pltpu
from jax.experimental.pallas import tpu_sc as plsc

N = 10000
NP = 10240
D = 128
TW = 144
E = 320000
NC = 2
NS = 16
EW = E // (NC * NS)
B = 80
NB = EW // B
RPT = NP // NS


def _prep_body(f_ref, wt_ref, b_ref, a2w_ref, a2b_ref, t_ref):
    h = jnp.dot(f_ref[...], wt_ref[...], preferred_element_type=jnp.float32)
    h = h + b_ref[...]
    a2 = jnp.sum(h * a2w_ref[...], axis=1, keepdims=True) + a2b_ref[0, 0]
    g = jnp.exp(a2)
    pad = jnp.zeros((h.shape[0], TW - D - 1), jnp.float32)
    t_ref[...] = jnp.concatenate([h * g, g, pad], axis=1)


def _make_table(features, wt, bvec, a2w, a2b):
    bm = 2000
    grid = (N // bm,)
    return pl.pallas_call(
        _prep_body,
        grid=grid,
        in_specs=[
            pl.BlockSpec((bm, D), lambda i: (i, 0)),
            pl.BlockSpec((D, D), lambda i: (0, 0)),
            pl.BlockSpec((1, D), lambda i: (0, 0)),
            pl.BlockSpec((1, D), lambda i: (0, 0)),
            pl.BlockSpec((1, 1), lambda i: (0, 0)),
        ],
        out_specs=pl.BlockSpec((bm, TW), lambda i: (i, 0)),
        out_shape=jax.ShapeDtypeStruct((NP, TW), jnp.float32),
    )(features, wt, bvec, a2w, a2b)


NQUAD = (NB - 1) // 4


def _edge_body(idx_hbm, t_hbm, out_hbm,
               ib0, ib1, ib2, ib3, r0, r1, acc,
               i0s, i1s, i2s, i3s, g0, g1, t0, t1):
    cid = lax.axis_index("c")
    sid = lax.axis_index("s")
    ib = (ib0, ib1, ib2, ib3)
    isem = (i0s, i1s, i2s, i3s)
    rows = (r0, r1)
    gsem = (g0, g1)
    ssem = (t0, t1)

    def zb(i, _):
        for k in range(TW // 16):
            r0[i, pl.ds(k * 16, 16)] = jnp.zeros((16,), jnp.float32)
        return 0

    lax.fori_loop(0, B, zb, 0)
    for j in range(RPT // B):
        pltpu.sync_copy(r0, acc.at[pl.ds(sid * RPT + j * B, B)])
    plsc.subcore_barrier()

    wbase = (cid * NS + sid) * NB

    def idx_start(bi, j):
        pltpu.async_copy(idx_hbm.at[wbase + bi], ib[j], isem[j])

    def idx_wait(j):
        pltpu.make_async_copy(idx_hbm.at[0], ib[j], isem[j]).wait()

    def gather_start(j, b):
        pltpu.async_copy(t_hbm.at[ib[j].at[1]], rows[b], gsem[b])

    def gather_wait(j, b):
        pltpu.make_async_copy(t_hbm.at[ib[j].at[1]], rows[b], gsem[b]).wait()

    def scatter_start(j, b):
        pltpu.async_copy(rows[b], acc.at[ib[j].at[0]], ssem[b], add=True)

    def scatter_wait(j, b):
        pltpu.make_async_copy(rows[b], acc.at[ib[j].at[0]], ssem[b]).wait()

    idx_start(0, 0)
    idx_start(1, 1)
    idx_wait(0)
    gather_start(0, 0)

    def quad(qi, _):
        i_base = qi * 4
        for k in range(4):
            i = i_base + k
            b = k % 2
            if k == 0:
                @pl.when(qi > 0)
                def _():
                    scatter_wait(3, 1)
            else:
                scatter_wait(k - 1, 1 - b)
            if k == 3:
                @pl.when(qi < NQUAD - 1)
                def _():
                    idx_start(i + 2, 1)
            else:
                idx_start(i + 2, (k + 2) % 4)
            idx_wait((k + 1) % 4)
            gather_start((k + 1) % 4, 1 - b)
            gather_wait(k, b)
            scatter_start(k, b)
        return 0

    lax.fori_loop(0, NQUAD, quad, 0)
    scatter_wait(3, 1)
    gather_wait(0, 0)
    pltpu.sync_copy(r0, acc.at[ib0.at[0]], add=True)

    plsc.subcore_barrier()
    pltpu.sync_copy(
        acc.at[pl.ds(sid * RPT, RPT)],
        out_hbm.at[cid, pl.ds(sid * RPT, RPT)],
    )


def _edge_accumulate(idx_packed, table):
    mesh = plsc.VectorSubcoreMesh(core_axis_name="c", subcore_axis_name="s")
    f = pl.kernel(
        _edge_body,
        out_type=jax.ShapeDtypeStruct((NC, NP, TW), jnp.float32),
        mesh=mesh,
        scratch_types=(
            [pltpu.VMEM((2, B), jnp.int32)] * 4
            + [pltpu.VMEM((B, TW), jnp.float32)] * 2
            + [pltpu.VMEM_SHARED((NP, TW), jnp.float32)]
            + [pltpu.SemaphoreType.DMA] * 8
        ),
        compiler_params=pltpu.CompilerParams(use_tc_tiling_on_sc=False),
    )
    return f(idx_packed, table)


def _comb_body(acc_ref, o_ref):
    a = acc_ref[0] + acc_ref[1]
    den = a[:, D:D + 1]
    den = jnp.where(den > 0.0, den, 1.0)
    o_ref[...] = a[:, :D] / den


def _combine(acc):
    bm = 2000
    grid = (N // bm,)
    return pl.pallas_call(
        _comb_body,
        grid=grid,
        in_specs=[pl.BlockSpec((NC, bm, TW), lambda i: (0, i, 0))],
        out_specs=pl.BlockSpec((bm, D), lambda i: (i, 0)),
        out_shape=jax.ShapeDtypeStruct((N, D), jnp.float32),
    )(acc)


@jax.jit
def kernel(features, edge_index, W, b, a1_w, a1_b, a2_w, a2_b):
    del a1_w, a1_b
    table = _make_table(
        features,
        W.T,
        b.reshape(1, D),
        a2_w.reshape(1, D),
        a2_b.reshape(1, 1),
    )
    idx_packed = edge_index.reshape(2, E // B, B).transpose(1, 0, 2)
    acc = _edge_accumulate(idx_packed, table)
    return _combine(acc)


# speedup vs baseline: 58.5071x; 1.1108x over previous
import functools

import jax
import jax.numpy as jnp
from jax import lax
from jax.experimental import pallas as pl
from jax.experimental.pallas import tpu as pltpu
from jax.experimental.pallas import tpu_sc as plsc

N = 10000
NP = 10240
D = 128
TW = 144
E = 320000
NC = 2
NS = 16
EW = E // (NC * NS)
B = 80
NB = EW // B
RPT = NP // NS


def _prep_body(f_ref, wt_ref, b_ref, a2w_ref, a2b_ref, t_ref):
    h = jnp.dot(f_ref[...], wt_ref[...], preferred_element_type=jnp.float32)
    h = h + b_ref[...]
    a2 = jnp.sum(h * a2w_ref[...], axis=1, keepdims=True) + a2b_ref[0, 0]
    g = jnp.exp(a2)
    pad = jnp.zeros((h.shape[0], TW - D - 1), jnp.float32)
    t_ref[...] = jnp.concatenate([h * g, g, pad], axis=1)


def _make_table(features, wt, bvec, a2w, a2b):
    bm = 2000
    grid = (N // bm,)
    return pl.pallas_call(
        _prep_body,
        grid=grid,
        in_specs=[
            pl.BlockSpec((bm, D), lambda i: (i, 0)),
            pl.BlockSpec((D, D), lambda i: (0, 0)),
            pl.BlockSpec((1, D), lambda i: (0, 0)),
            pl.BlockSpec((1, D), lambda i: (0, 0)),
            pl.BlockSpec((1, 1), lambda i: (0, 0)),
        ],
        out_specs=pl.BlockSpec((bm, TW), lambda i: (i, 0)),
        out_shape=jax.ShapeDtypeStruct((NP, TW), jnp.float32),
    )(features, wt, bvec, a2w, a2b)


NQUAD = (NB - 1) // 4


def _edge_body(idx_hbm, t_hbm, out_hbm,
               ib0, ib1, ib2, ib3, r0, r1, acc,
               i0s, i1s, i2s, i3s, g0, g1, t0, t1):
    cid = lax.axis_index("c")
    sid = lax.axis_index("s")
    ib = (ib0, ib1, ib2, ib3)
    isem = (i0s, i1s, i2s, i3s)
    rows = (r0, r1)
    gsem = (g0, g1)
    ssem = (t0, t1)

    def zb(i, _):
        for k in range(TW // 16):
            r0[i, pl.ds(k * 16, 16)] = jnp.zeros((16,), jnp.float32)
        return 0

    lax.fori_loop(0, B, zb, 0)
    for j in range(RPT // B):
        pltpu.sync_copy(r0, acc.at[pl.ds(sid * RPT + j * B, B)])
    plsc.subcore_barrier()

    ebase = (cid * NS + sid) * EW

    def idx_start(bi, j):
        pltpu.async_copy(idx_hbm.at[:, pl.ds(ebase + bi * B, B)], ib[j], isem[j])

    def idx_wait(j):
        pltpu.make_async_copy(idx_hbm.at[:, pl.ds(0, B)], ib[j], isem[j]).wait()

    def gather_start(j, b):
        pltpu.async_copy(t_hbm.at[ib[j].at[1]], rows[b], gsem[b])

    def gather_wait(j, b):
        pltpu.make_async_copy(t_hbm.at[ib[j].at[1]], rows[b], gsem[b]).wait()

    def scatter_start(j, b):
        pltpu.async_copy(rows[b], acc.at[ib[j].at[0]], ssem[b], add=True)

    def scatter_wait(j, b):
        pltpu.make_async_copy(rows[b], acc.at[ib[j].at[0]], ssem[b]).wait()

    idx_start(0, 0)
    idx_start(1, 1)
    idx_wait(0)
    gather_start(0, 0)

    def quad(qi, _):
        i_base = qi * 4
        for k in range(4):
            i = i_base + k
            b = k % 2
            if k == 0:
                @pl.when(qi > 0)
                def _():
                    scatter_wait(3, 1)
            else:
                scatter_wait(k - 1, 1 - b)
            if k == 3:
                @pl.when(qi < NQUAD - 1)
                def _():
                    idx_start(i + 2, 1)
            else:
                idx_start(i + 2, (k + 2) % 4)
            idx_wait((k + 1) % 4)
            gather_start((k + 1) % 4, 1 - b)
            gather_wait(k, b)
            scatter_start(k, b)
        return 0

    lax.fori_loop(0, NQUAD, quad, 0)
    scatter_wait(3, 1)
    gather_wait(0, 0)
    pltpu.sync_copy(r0, acc.at[ib0.at[0]], add=True)

    plsc.subcore_barrier()
    pltpu.sync_copy(
        acc.at[pl.ds(sid * RPT, RPT)],
        out_hbm.at[cid, pl.ds(sid * RPT, RPT)],
    )


def _edge_accumulate(edge_index, table):
    mesh = plsc.VectorSubcoreMesh(core_axis_name="c", subcore_axis_name="s")
    f = pl.kernel(
        _edge_body,
        out_type=jax.ShapeDtypeStruct((NC, NP, TW), jnp.float32),
        mesh=mesh,
        scratch_types=(
            [pltpu.VMEM((2, B), jnp.int32)] * 4
            + [pltpu.VMEM((B, TW), jnp.float32)] * 2
            + [pltpu.VMEM_SHARED((NP, TW), jnp.float32)]
            + [pltpu.SemaphoreType.DMA] * 8
        ),
        compiler_params=pltpu.CompilerParams(use_tc_tiling_on_sc=False),
    )
    return f(edge_index, table)


def _comb_body(acc_ref, o_ref):
    a = acc_ref[0] + acc_ref[1]
    den = a[:, D:D + 1]
    den = jnp.where(den > 0.0, den, 1.0)
    o_ref[...] = a[:, :D] / den


def _combine(acc):
    bm = 2000
    grid = (N // bm,)
    return pl.pallas_call(
        _comb_body,
        grid=grid,
        in_specs=[pl.BlockSpec((NC, bm, TW), lambda i: (0, i, 0))],
        out_specs=pl.BlockSpec((bm, D), lambda i: (i, 0)),
        out_shape=jax.ShapeDtypeStruct((N, D), jnp.float32),
    )(acc)


@jax.jit
def kernel(features, edge_index, W, b, a1_w, a1_b, a2_w, a2_b):
    del a1_w, a1_b
    table = _make_table(
        features,
        W.T,
        b.reshape(1, D),
        a2_w.reshape(1, D),
        a2_b.reshape(1, 1),
    )
    acc = _edge_accumulate(edge_index, table)
    return _combine(acc)


# speedup vs baseline: 70.8294x; 1.2106x over previous
import jax
import jax.numpy as jnp
from jax import lax
from jax.experimental import pallas as pl
from jax.experimental.pallas import tpu as pltpu
from jax.experimental.pallas import tpu_sc as plsc

N = 10000
NP = 10240
D = 128
E = 320000
NC = 2
NS = 16
NW = NC * NS
EW = E // NW
B = 80
NB = EW // B
RPT = NP // NS


def _prep_body(f_ref, wt_ref, b_ref, a2w_ref, a2b_ref, t_ref, gv_ref):
    h = jnp.dot(f_ref[...], wt_ref[...], preferred_element_type=jnp.float32)
    h = h + b_ref[...]
    a2 = jnp.sum(h * a2w_ref[...], axis=1, keepdims=True) + a2b_ref[0, 0]
    g = jnp.exp(a2)
    t_ref[...] = h * g
    gv_ref[...] = jnp.transpose(g)


def _make_table(features, wt, bvec, a2w, a2b):
    bm = 2048
    grid = (NP // bm,)
    return pl.pallas_call(
        _prep_body,
        grid=grid,
        in_specs=[
            pl.BlockSpec((bm, D), lambda i: (i, 0)),
            pl.BlockSpec((D, D), lambda i: (0, 0)),
            pl.BlockSpec((1, D), lambda i: (0, 0)),
            pl.BlockSpec((1, D), lambda i: (0, 0)),
            pl.BlockSpec((1, 1), lambda i: (0, 0)),
        ],
        out_specs=[
            pl.BlockSpec((bm, D), lambda i: (i, 0)),
            pl.BlockSpec((1, bm), lambda i: (0, i)),
        ],
        out_shape=[
            jax.ShapeDtypeStruct((NP, D), jnp.float32),
            jax.ShapeDtypeStruct((1, NP), jnp.float32),
        ],
    )(features, wt, bvec, a2w, a2b)


NQUAD = (NB - 1) // 4


def _edge_body(idx_hbm, t_hbm, gv_hbm, out_hbm, rs_hbm,
               ib0, ib1, ib2, ib3, r0, r1, gv, rs, acc,
               i0s, i1s, i2s, i3s, g0, g1, t0, t1):
    cid = lax.axis_index("c")
    sid = lax.axis_index("s")
    ib = (ib0, ib1, ib2, ib3)
    isem = (i0s, i1s, i2s, i3s)
    rows = (r0, r1)
    gsem = (g0, g1)
    ssem = (t0, t1)

    pltpu.sync_copy(gv_hbm.at[0], gv)

    def zrs(i, _):
        rs[pl.ds(i * 16, 16)] = jnp.zeros((16,), jnp.float32)
        return 0

    lax.fori_loop(0, NP // 16, zrs, 0)

    def zb(i, _):
        for k in range(D // 16):
            r0[i, pl.ds(k * 16, 16)] = jnp.zeros((16,), jnp.float32)
        return 0

    lax.fori_loop(0, B, zb, 0)
    for j in range(RPT // B):
        pltpu.sync_copy(r0, acc.at[pl.ds(sid * RPT + j * B, B)])
    plsc.subcore_barrier()

    ebase = (cid * NS + sid) * EW

    def idx_start(bi, j):
        pltpu.async_copy(idx_hbm.at[:, pl.ds(ebase + bi * B, B)], ib[j], isem[j])

    def idx_wait(j):
        pltpu.make_async_copy(idx_hbm.at[:, pl.ds(0, B)], ib[j], isem[j]).wait()

    def gather_start(j, b):
        pltpu.async_copy(t_hbm.at[ib[j].at[1]], rows[b], gsem[b])

    def gather_wait(j, b):
        pltpu.make_async_copy(t_hbm.at[ib[j].at[1]], rows[b], gsem[b]).wait()

    def scatter_start(j, b):
        pltpu.async_copy(rows[b], acc.at[ib[j].at[0]], ssem[b], add=True)

    def scatter_wait(j, b):
        pltpu.make_async_copy(rows[b], acc.at[ib[j].at[0]], ssem[b]).wait()

    def rowsum_batch(j):
        for g5 in range(B // 16):
            dst16 = ib[j][1, pl.ds(g5 * 16, 16)]
            src16 = ib[j][0, pl.ds(g5 * 16, 16)]
            gvals = plsc.load_gather(gv, [dst16])
            plsc.addupdate_scatter(rs, [src16], gvals)

    idx_start(0, 0)
    idx_start(1, 1)
    idx_wait(0)
    gather_start(0, 0)

    def quad(qi, _):
        i_base = qi * 4
        for k in range(4):
            i = i_base + k
            b = k % 2
            if k == 0:
                @pl.when(qi > 0)
                def _():
                    scatter_wait(3, 1)
            else:
                scatter_wait(k - 1, 1 - b)
            if k == 3:
                @pl.when(qi < NQUAD - 1)
                def _():
                    idx_start(i + 2, 1)
            else:
                idx_start(i + 2, (k + 2) % 4)
            idx_wait((k + 1) % 4)
            gather_start((k + 1) % 4, 1 - b)
            rowsum_batch(k)
            gather_wait(k, b)
            scatter_start(k, b)
        return 0

    lax.fori_loop(0, NQUAD, quad, 0)
    scatter_wait(3, 1)
    rowsum_batch(0)
    gather_wait(0, 0)
    pltpu.sync_copy(r0, acc.at[ib0.at[0]], add=True)

    plsc.subcore_barrier()
    pltpu.sync_copy(
        acc.at[pl.ds(sid * RPT, RPT)],
        out_hbm.at[cid, pl.ds(sid * RPT, RPT)],
    )
    pltpu.sync_copy(rs, rs_hbm.at[cid * NS + sid])


def _edge_accumulate(edge_index, table, gvec):
    mesh = plsc.VectorSubcoreMesh(core_axis_name="c", subcore_axis_name="s")
    f = pl.kernel(
        _edge_body,
        out_type=[
            jax.ShapeDtypeStruct((NC, NP, D), jnp.float32),
            jax.ShapeDtypeStruct((NW, NP), jnp.float32),
        ],
        mesh=mesh,
        scratch_types=(
            [pltpu.VMEM((2, B), jnp.int32)] * 4
            + [pltpu.VMEM((B, D), jnp.float32)] * 2
            + [pltpu.VMEM((NP,), jnp.float32)] * 2
            + [pltpu.VMEM_SHARED((NP, D), jnp.float32)]
            + [pltpu.SemaphoreType.DMA] * 8
        ),
        compiler_params=pltpu.CompilerParams(
            use_tc_tiling_on_sc=False, needs_layout_passes=False
        ),
    )
    return f(edge_index, table, gvec)


def _comb_body(acc_ref, rs_ref, o_ref):
    a = acc_ref[0] + acc_ref[1]
    den = jnp.sum(rs_ref[...], axis=0, keepdims=True)
    den = jnp.where(den > 0.0, den, 1.0)
    o_ref[...] = a / jnp.transpose(den)


def _combine(acc, rsum):
    bm = 2048
    grid = (NP // bm,)
    return pl.pallas_call(
        _comb_body,
        grid=grid,
        in_specs=[
            pl.BlockSpec((NC, bm, D), lambda i: (0, i, 0)),
            pl.BlockSpec((NW, bm), lambda i: (0, i)),
        ],
        out_specs=pl.BlockSpec((bm, D), lambda i: (i, 0)),
        out_shape=jax.ShapeDtypeStruct((N, D), jnp.float32),
    )(acc, rsum)


@jax.jit
def kernel(features, edge_index, W, b, a1_w, a1_b, a2_w, a2_b):
    del a1_w, a1_b
    table, gvec = _make_table(
        features,
        W.T,
        b.reshape(1, D),
        a2_w.reshape(1, D),
        a2_b.reshape(1, 1),
    )
    acc, rsum = _edge_accumulate(edge_index, table, gvec)
    return _combine(acc, rsum)


# speedup vs baseline: 71.6923x; 1.0122x over previous
import jax
import jax.numpy as jnp
from jax import lax
from jax.experimental import pallas as pl
from jax.experimental.pallas import tpu as pltpu
from jax.experimental.pallas import tpu_sc as plsc

N = 10000
NP = 10240
D = 128
E = 320000
NC = 2
NS = 16
NW = NC * NS
EW = E // NW
B = 80
NB = EW // B
RPT = NP // NS


def _prep_body(f_ref, w_ref, b_ref, a2w_ref, a2b_ref, t_ref, gv_ref):
    h = lax.dot_general(
        f_ref[...], w_ref[...], (((1,), (1,)), ((), ())),
        preferred_element_type=jnp.float32,
    )
    h = h + b_ref[...]
    a2 = jnp.sum(h * a2w_ref[...], axis=1, keepdims=True) + a2b_ref[0, 0]
    g = jnp.exp(a2)
    t_ref[...] = h * g
    gv_ref[...] = jnp.transpose(g)


def _make_table(features, wt, bvec, a2w, a2b):
    bm = 2048
    grid = (NP // bm,)
    return pl.pallas_call(
        _prep_body,
        grid=grid,
        in_specs=[
            pl.BlockSpec((bm, D), lambda i: (i, 0)),
            pl.BlockSpec((D, D), lambda i: (0, 0)),
            pl.BlockSpec((1, D), lambda i: (0, 0)),
            pl.BlockSpec((1, D), lambda i: (0, 0)),
            pl.BlockSpec((1, 1), lambda i: (0, 0)),
        ],
        out_specs=[
            pl.BlockSpec((bm, D), lambda i: (i, 0)),
            pl.BlockSpec((1, bm), lambda i: (0, i)),
        ],
        out_shape=[
            jax.ShapeDtypeStruct((NP, D), jnp.float32),
            jax.ShapeDtypeStruct((1, NP), jnp.float32),
        ],
    )(features, wt, bvec, a2w, a2b)


NQUAD = (NB - 1) // 4


def _edge_body(idx_hbm, t_hbm, gv_hbm, out_hbm, rs_hbm,
               ib0, ib1, ib2, ib3, r0, r1, gv, rs, acc,
               i0s, i1s, i2s, i3s, g0, g1, t0, t1):
    cid = lax.axis_index("c")
    sid = lax.axis_index("s")
    ib = (ib0, ib1, ib2, ib3)
    isem = (i0s, i1s, i2s, i3s)
    rows = (r0, r1)
    gsem = (g0, g1)
    ssem = (t0, t1)

    pltpu.sync_copy(gv_hbm.at[0], gv)

    def zrs(i, _):
        rs[pl.ds(i * 16, 16)] = jnp.zeros((16,), jnp.float32)
        return 0

    lax.fori_loop(0, NP // 16, zrs, 0)

    def zb(i, _):
        for k in range(D // 16):
            r0[i, pl.ds(k * 16, 16)] = jnp.zeros((16,), jnp.float32)
        return 0

    lax.fori_loop(0, B, zb, 0)
    for j in range(RPT // B):
        pltpu.sync_copy(r0, acc.at[pl.ds(sid * RPT + j * B, B)])
    plsc.subcore_barrier()

    ebase = (cid * NS + sid) * EW

    def idx_start(bi, j):
        pltpu.async_copy(idx_hbm.at[:, pl.ds(ebase + bi * B, B)], ib[j], isem[j])

    def idx_wait(j):
        pltpu.make_async_copy(idx_hbm.at[:, pl.ds(0, B)], ib[j], isem[j]).wait()

    def gather_start(j, b):
        pltpu.async_copy(t_hbm.at[ib[j].at[1]], rows[b], gsem[b])

    def gather_wait(j, b):
        pltpu.make_async_copy(t_hbm.at[ib[j].at[1]], rows[b], gsem[b]).wait()

    def scatter_start(j, b):
        pltpu.async_copy(rows[b], acc.at[ib[j].at[0]], ssem[b], add=True)

    def scatter_wait(j, b):
        pltpu.make_async_copy(rows[b], acc.at[ib[j].at[0]], ssem[b]).wait()

    def rowsum_batch(j):
        for g5 in range(B // 16):
            dst16 = ib[j][1, pl.ds(g5 * 16, 16)]
            src16 = ib[j][0, pl.ds(g5 * 16, 16)]
            gvals = plsc.load_gather(gv, [dst16])
            plsc.addupdate_scatter(rs, [src16], gvals)

    idx_start(0, 0)
    idx_start(1, 1)
    idx_wait(0)
    gather_start(0, 0)

    def quad(qi, _):
        i_base = qi * 4
        for k in range(4):
            i = i_base + k
            b = k % 2
            if k == 0:
                @pl.when(qi > 0)
                def _():
                    scatter_wait(3, 1)
            else:
                scatter_wait(k - 1, 1 - b)
            if k == 3:
                @pl.when(qi < NQUAD - 1)
                def _():
                    idx_start(i + 2, 1)
            else:
                idx_start(i + 2, (k + 2) % 4)
            idx_wait((k + 1) % 4)
            gather_start((k + 1) % 4, 1 - b)
            rowsum_batch(k)
            gather_wait(k, b)
            scatter_start(k, b)
        return 0

    lax.fori_loop(0, NQUAD, quad, 0)
    scatter_wait(3, 1)
    rowsum_batch(0)
    gather_wait(0, 0)
    pltpu.sync_copy(r0, acc.at[ib0.at[0]], add=True)

    plsc.subcore_barrier()
    pltpu.sync_copy(
        acc.at[pl.ds(sid * RPT, RPT)],
        out_hbm.at[cid, pl.ds(sid * RPT, RPT)],
    )
    pltpu.sync_copy(rs, rs_hbm.at[cid * NS + sid])


def _edge_accumulate(edge_index, table, gvec):
    mesh = plsc.VectorSubcoreMesh(core_axis_name="c", subcore_axis_name="s")
    f = pl.kernel(
        _edge_body,
        out_type=[
            jax.ShapeDtypeStruct((NC, NP, D), jnp.float32),
            jax.ShapeDtypeStruct((NW, NP), jnp.float32),
        ],
        mesh=mesh,
        scratch_types=(
            [pltpu.VMEM((2, B), jnp.int32)] * 4
            + [pltpu.VMEM((B, D), jnp.float32)] * 2
            + [pltpu.VMEM((NP,), jnp.float32)] * 2
            + [pltpu.VMEM_SHARED((NP, D), jnp.float32)]
            + [pltpu.SemaphoreType.DMA] * 8
        ),
        compiler_params=pltpu.CompilerParams(
            use_tc_tiling_on_sc=False, needs_layout_passes=False
        ),
    )
    return f(edge_index, table, gvec)


def _comb_body(acc_ref, rs_ref, o_ref):
    a = acc_ref[0] + acc_ref[1]
    den = jnp.sum(rs_ref[...], axis=0, keepdims=True)
    den = jnp.where(den > 0.0, den, 1.0)
    o_ref[...] = a / jnp.transpose(den)


def _combine(acc, rsum):
    bm = 2048
    grid = (NP // bm,)
    return pl.pallas_call(
        _comb_body,
        grid=grid,
        in_specs=[
            pl.BlockSpec((NC, bm, D), lambda i: (0, i, 0)),
            pl.BlockSpec((NW, bm), lambda i: (0, i)),
        ],
        out_specs=pl.BlockSpec((bm, D), lambda i: (i, 0)),
        out_shape=jax.ShapeDtypeStruct((N, D), jnp.float32),
    )(acc, rsum)


@jax.jit
def kernel(features, edge_index, W, b, a1_w, a1_b, a2_w, a2_b):
    del a1_w, a1_b
    table, gvec = _make_table(
        features,
        W,
        b.reshape(1, D),
        a2_w.reshape(1, D),
        a2_b.reshape(1, 1),
    )
    acc, rsum = _edge_accumulate(edge_index, table, gvec)
    return _combine(acc, rsum)
